# baseline (speedup 1.0000x reference)
import functools

import jax
import jax.numpy as jnp
import numpy as np
from jax import lax
from jax.experimental import pallas as pl
from jax.experimental.pallas import tpu as pltpu
from jax.experimental.pallas import tpu_sc as plsc

N = 10000
E = 320000
D = 128
H = 128
C = 16
L = 4

NC = 2
NS = 16
NW = NC * NS
CPT = H // NW
CHK = 4096
NCH = 80
EP = NCH * CHK
NB = 10240
GRP = CHK // 16

_MESH = plsc.VectorSubcoreMesh(core_axis_name="c", subcore_axis_name="s")
_SC_PARAMS = pltpu.CompilerParams(needs_layout_passes=False)


def _agg_body(h_hbm, pk_hbm, h_sl, acc, deg, ebuf, sem0, sem1, with_deg):
    c = lax.axis_index("c")
    s = lax.axis_index("s")
    wid = c * jnp.int32(NS) + s
    row0 = wid * jnp.int32(CPT)
    pltpu.async_copy(h_hbm.at[pl.ds(row0, CPT)], h_sl, sem1)
    pltpu.async_copy(pk_hbm.at[jnp.int32(0)], ebuf.at[np.int32(0)], sem0)

    zero16 = jnp.zeros((16,), jnp.float32)

    @plsc.parallel_loop(jnp.int32(0), jnp.int32(NB // 16),
                        jnp.int32(1), unroll=4)
    def _(j):
        o = j * jnp.int32(16)
        for cc in range(CPT):
            acc[np.int32(cc), pl.ds(o, 16)] = zero16
        if with_deg:
            deg[pl.ds(o, 16)] = zero16

    pltpu.make_async_copy(h_hbm.at[pl.ds(row0, CPT)], h_sl, sem1).wait()

    sems = (sem0, sem1)
    ones16 = jnp.ones((16,), jnp.float32)
    last = jnp.int32(NCH - 1)

    ccv = [jnp.full((16,), cc, jnp.int32) for cc in range(CPT)]

    def grp16(b, base):
        srcv = ebuf[np.int32(b), np.int32(0), pl.ds(base, 16)]
        dstv = ebuf[np.int32(b), np.int32(1), pl.ds(base, 16)]
        for cc in range(CPT):
            vals = plsc.load_gather(h_sl, [ccv[cc], srcv])
            plsc.addupdate_scatter(acc, [ccv[cc], dstv], vals)
        if with_deg:
            plsc.addupdate_scatter(deg, [dstv], ones16)

    def pair(g, carry):
        for b in range(2):
            i = g * jnp.int32(2) + jnp.int32(b)
            nxt = jnp.minimum(i + jnp.int32(1), last)
            pltpu.async_copy(pk_hbm.at[nxt], ebuf.at[np.int32(1 - b)],
                             sems[1 - b])
            pltpu.make_async_copy(pk_hbm.at[i], ebuf.at[np.int32(b)],
                                  sems[b]).wait()

            @plsc.parallel_loop(jnp.int32(0), jnp.int32(GRP),
                                jnp.int32(1), unroll=8)
            def _(j):
                grp16(b, j * jnp.int32(16))
        return carry

    lax.fori_loop(jnp.int32(0), jnp.int32(NCH // 2), pair, jnp.int32(0))
    pltpu.make_async_copy(pk_hbm.at[last], ebuf.at[np.int32(0)], sem0).wait()
    return row0, wid


@functools.partial(
    pl.kernel,
    out_type=jax.ShapeDtypeStruct((NW * CPT, NB), jnp.float32),
    mesh=_MESH,
    compiler_params=_SC_PARAMS,
    scratch_types=[
        pltpu.VMEM((CPT, NB), jnp.float32),
        pltpu.VMEM((CPT, NB), jnp.float32),
        pltpu.VMEM((2, 2, CHK), jnp.int32),
        pltpu.SemaphoreType.DMA,
        pltpu.SemaphoreType.DMA,
    ],
)
def _sc_agg(h_hbm, pk_hbm, out_hbm, h_sl, acc, ebuf, sem0, sem1):
    row0, _ = _agg_body(h_hbm, pk_hbm, h_sl, acc, None, ebuf, sem0, sem1,
                        False)
    pltpu.sync_copy(acc, out_hbm.at[pl.ds(row0, CPT)])


@functools.partial(
    pl.kernel,
    out_type=(jax.ShapeDtypeStruct((NW * CPT, NB), jnp.float32),
              jax.ShapeDtypeStruct((NB,), jnp.float32)),
    mesh=_MESH,
    compiler_params=_SC_PARAMS,
    scratch_types=[
        pltpu.VMEM((CPT, NB), jnp.float32),
        pltpu.VMEM((CPT, NB), jnp.float32),
        pltpu.VMEM((NB,), jnp.float32),
        pltpu.VMEM((2, 2, CHK), jnp.int32),
        pltpu.SemaphoreType.DMA,
        pltpu.SemaphoreType.DMA,
    ],
)
def _sc_agg_deg(h_hbm, pk_hbm, out_hbm, deg_hbm, h_sl, acc, deg, ebuf,
                sem0, sem1):
    row0, wid = _agg_body(h_hbm, pk_hbm, h_sl, acc, deg, ebuf, sem0, sem1,
                          True)
    pltpu.sync_copy(acc, out_hbm.at[pl.ds(row0, CPT)])

    @pl.when(wid == jnp.int32(0))
    def _():
        pltpu.sync_copy(deg, deg_hbm)


_Z = np.int32(0)
R = 2048
G = NB // R


def _dot(a, b):
    return jnp.dot(a, b, preferred_element_type=jnp.float32,
                   precision=lax.Precision.HIGHEST)


def _tc_encode(h_ref, w_ref, b_ref, out_ref):
    out_ref[...] = lax.dot_general(
        w_ref[...], h_ref[...], (((1,), (1,)), ((), ())),
        preferred_element_type=jnp.float32,
        precision=lax.Precision.HIGHEST) + b_ref[...]


def _tc_mm1(h_ref, agg_ref, deg_ref, wt_ref, wb_ref, b_ref,
            hl_ref, degs_ref, sum_ref, sq_ref):
    degs = jnp.maximum(deg_ref[...], 1.0)
    degs_ref[...] = degs
    aggn = agg_ref[...] * (1.0 / degs)
    hl = _dot(wt_ref[...], h_ref[...]) + _dot(wb_ref[...], aggn) + b_ref[...]
    hl_ref[...] = hl

    @pl.when(pl.program_id(0) == 0)
    def _():
        sum_ref[...] = jnp.zeros_like(sum_ref)
        sq_ref[...] = jnp.zeros_like(sq_ref)

    col0 = pl.program_id(0) * R
    lane = lax.broadcasted_iota(jnp.int32, (1, R), 1) + col0
    hv = hl * (lane < N).astype(jnp.float32)
    sum_ref[...] += jnp.sum(hv, axis=1, keepdims=True)
    sq_ref[...] += jnp.sum(hv * hv, axis=1, keepdims=True)


def _tc_mm(h_ref, agg_ref, degs_ref, wt_ref, wb_ref, b_ref,
           hl_ref, sum_ref, sq_ref):
    aggn = agg_ref[...] * (1.0 / degs_ref[...])
    hl = _dot(wt_ref[...], h_ref[...]) + _dot(wb_ref[...], aggn) + b_ref[...]
    hl_ref[...] = hl

    @pl.when(pl.program_id(0) == 0)
    def _():
        sum_ref[...] = jnp.zeros_like(sum_ref)
        sq_ref[...] = jnp.zeros_like(sq_ref)

    col0 = pl.program_id(0) * R
    lane = lax.broadcasted_iota(jnp.int32, (1, R), 1) + col0
    hv = hl * (lane < N).astype(jnp.float32)
    sum_ref[...] += jnp.sum(hv, axis=1, keepdims=True)
    sq_ref[...] += jnp.sum(hv * hv, axis=1, keepdims=True)


def _bn(hl, sm, sq, g, be):
    mu = sm * (1.0 / N)
    var = sq * (1.0 / N) - mu * mu
    return (hl - mu) * lax.rsqrt(var + 1e-5) * g + be


def _tc_norm(hl_ref, sum_ref, sq_ref, degs_ref, g_ref, be_ref, out_ref):
    hn = _bn(hl_ref[...], sum_ref[...], sq_ref[...], g_ref[...], be_ref[...])
    out_ref[...] = jnp.maximum(hn, 0.0) * lax.rsqrt(degs_ref[...])


def _tc_norm_out(hl_ref, sum_ref, sq_ref, degs_ref, g_ref, be_ref,
                 wo_ref, bo_ref, out_ref):
    hn = _bn(hl_ref[...], sum_ref[...], sq_ref[...], g_ref[...], be_ref[...])
    hr = jnp.maximum(hn, 0.0) * lax.rsqrt(degs_ref[...])
    out_ref[...] = _dot(wo_ref[...], hr) + bo_ref[...]


def kernel(h, edge_index, e, W_enc, b_enc, W_layers, b_layers, gamma, beta,
           W_out, b_out):
    f32 = jnp.float32
    src = edge_index[0].astype(jnp.int32)
    dst = edge_index[1].astype(jnp.int32)
    srcp = jnp.pad(src, (0, EP - E)).reshape(NCH, CHK)
    dstp = jnp.pad(dst, (0, EP - E),
                   constant_values=NB - 1).reshape(NCH, CHK)
    packed = jnp.stack([srcp, dstp], axis=1)

    colsT = pl.BlockSpec((H, R), lambda i: (_Z, i))
    rows1T = pl.BlockSpec((1, R), lambda i: (_Z, i))
    const = pl.BlockSpec((H, H), lambda i: (_Z, _Z))
    ccol = pl.BlockSpec((H, 1), lambda i: (_Z, _Z))
    hrows = pl.BlockSpec((R, D), lambda i: (i, _Z))
    cwenc = pl.BlockSpec((H, D), lambda i: (_Z, _Z))
    cwo = pl.BlockSpec((C, H), lambda i: (_Z, _Z))
    cbo = pl.BlockSpec((C, 1), lambda i: (_Z, _Z))

    h = h.astype(f32)
    hT = pl.pallas_call(
        _tc_encode,
        grid=(G,),
        in_specs=[hrows, cwenc, ccol],
        out_specs=colsT,
        out_shape=jax.ShapeDtypeStruct((H, NB), f32),
    )(h, W_enc.astype(f32).T, b_enc.astype(f32).reshape(H, 1))

    wts = [(W_layers[i, :H, :].astype(f32).T, W_layers[i, H:, :].astype(f32).T,
            b_layers[i].astype(f32).reshape(H, 1),
            gamma[i].astype(f32).reshape(H, 1),
            beta[i].astype(f32).reshape(H, 1)) for i in range(L)]

    aggT, degTf = _sc_agg_deg(hT, packed)
    degT = degTf.reshape(1, NB)
    wt, wb, b, g, be = wts[0]
    hl, degs, sm, sq = pl.pallas_call(
        _tc_mm1,
        grid=(G,),
        in_specs=[colsT, colsT, rows1T, const, const, ccol],
        out_specs=(colsT, rows1T, ccol, ccol),
        out_shape=(jax.ShapeDtypeStruct((H, NB), f32),
                   jax.ShapeDtypeStruct((1, NB), f32),
                   jax.ShapeDtypeStruct((H, 1), f32),
                   jax.ShapeDtypeStruct((H, 1), f32)),
    )(hT, aggT, degT, wt, wb, b)
    hcur = pl.pallas_call(
        _tc_norm,
        grid=(G,),
        in_specs=[colsT, ccol, ccol, rows1T, ccol, ccol],
        out_specs=colsT,
        out_shape=jax.ShapeDtypeStruct((H, NB), f32),
    )(hl, sm, sq, degs, g, be)

    for i in range(1, L):
        aggT = _sc_agg(hcur, packed)
        wt, wb, b, g, be = wts[i]
        hl, sm, sq = pl.pallas_call(
            _tc_mm,
            grid=(G,),
            in_specs=[colsT, colsT, rows1T, const, const, ccol],
            out_specs=(colsT, ccol, ccol),
            out_shape=(jax.ShapeDtypeStruct((H, NB), f32),
                       jax.ShapeDtypeStruct((H, 1), f32),
                       jax.ShapeDtypeStruct((H, 1), f32)),
        )(hcur, aggT, degs, wt, wb, b)
        if i < L - 1:
            hcur = pl.pallas_call(
                _tc_norm,
                grid=(G,),
                in_specs=[colsT, ccol, ccol, rows1T, ccol, ccol],
                out_specs=colsT,
                out_shape=jax.ShapeDtypeStruct((H, NB), f32),
            )(hl, sm, sq, degs, g, be)
        else:
            outT = pl.pallas_call(
                _tc_norm_out,
                grid=(G,),
                in_specs=[colsT, ccol, ccol, rows1T, ccol, ccol, cwo, cbo],
                out_specs=pl.BlockSpec((C, R), lambda i: (_Z, i)),
                out_shape=jax.ShapeDtypeStruct((C, NB), f32),
            )(hl, sm, sq, degs, g, be,
              W_out.astype(f32).T, b_out.astype(f32).reshape(C, 1))
    return outT[:, :N].T.astype(jnp.float64)


# speedup vs baseline: 1.0747x; 1.0747x over previous
import functools

import jax
import jax.numpy as jnp
import numpy as np
from jax import lax
from jax.experimental import pallas as pl
from jax.experimental.pallas import tpu as pltpu
from jax.experimental.pallas import tpu_sc as plsc

N = 10000
E = 320000
D = 128
H = 128
C = 16
L = 4

NC = 2
NS = 16
NW = NC * NS
CPT = H // NW
CHK = 2048
NCH = 158
EP = NCH * CHK
NB = 10240
GRP = CHK // 16

_MESH = plsc.VectorSubcoreMesh(core_axis_name="c", subcore_axis_name="s")
_SC_PARAMS = pltpu.CompilerParams(needs_layout_passes=False)


def _agg_body(h_hbm, pk_hbm, h_sl, acc, deg, ebuf, sem0, sem1, with_deg):
    c = lax.axis_index("c")
    s = lax.axis_index("s")
    wid = c * jnp.int32(NS) + s
    row0 = wid * jnp.int32(CPT)
    pltpu.sync_copy(h_hbm.at[pl.ds(row0, CPT)], h_sl)

    zero16 = jnp.zeros((16,), jnp.float32)

    @plsc.parallel_loop(jnp.int32(0), jnp.int32(NB // 16),
                        jnp.int32(1), unroll=4)
    def _(j):
        o = j * jnp.int32(16)
        for cc in range(CPT):
            acc[np.int32(cc), pl.ds(o, 16)] = zero16
        if with_deg:
            deg[pl.ds(o, 16)] = zero16

    sems = (sem0, sem1)
    ones16 = jnp.ones((16,), jnp.float32)
    last = jnp.int32(NCH - 1)

    pltpu.async_copy(pk_hbm.at[jnp.int32(0)], ebuf.at[np.int32(0)], sem0)

    ccv = [jnp.full((16,), cc, jnp.int32) for cc in range(CPT)]

    def grp16(b, base):
        srcv = ebuf[np.int32(b), np.int32(0), pl.ds(base, 16)]
        dstv = ebuf[np.int32(b), np.int32(1), pl.ds(base, 16)]
        for cc in range(CPT):
            vals = plsc.load_gather(h_sl, [ccv[cc], srcv])
            plsc.addupdate_scatter(acc, [ccv[cc], dstv], vals)
        if with_deg:
            plsc.addupdate_scatter(deg, [dstv], ones16)

    def pair(g, carry):
        for b in range(2):
            i = g * jnp.int32(2) + jnp.int32(b)
            nxt = jnp.minimum(i + jnp.int32(1), last)
            pltpu.async_copy(pk_hbm.at[nxt], ebuf.at[np.int32(1 - b)],
                             sems[1 - b])
            pltpu.make_async_copy(pk_hbm.at[i], ebuf.at[np.int32(b)],
                                  sems[b]).wait()

            @plsc.parallel_loop(jnp.int32(0), jnp.int32(GRP),
                                jnp.int32(1), unroll=8)
            def _(j):
                grp16(b, j * jnp.int32(16))
        return carry

    lax.fori_loop(jnp.int32(0), jnp.int32(NCH // 2), pair, jnp.int32(0))
    pltpu.make_async_copy(pk_hbm.at[last], ebuf.at[np.int32(0)], sem0).wait()
    return row0, wid


@functools.partial(
    pl.kernel,
    out_type=jax.ShapeDtypeStruct((NW * CPT, NB), jnp.float32),
    mesh=_MESH,
    compiler_params=_SC_PARAMS,
    scratch_types=[
        pltpu.VMEM((CPT, NB), jnp.float32),
        pltpu.VMEM((CPT, NB), jnp.float32),
        pltpu.VMEM((2, 2, CHK), jnp.int32),
        pltpu.SemaphoreType.DMA,
        pltpu.SemaphoreType.DMA,
    ],
)
def _sc_agg(h_hbm, pk_hbm, out_hbm, h_sl, acc, ebuf, sem0, sem1):
    row0, _ = _agg_body(h_hbm, pk_hbm, h_sl, acc, None, ebuf, sem0, sem1,
                        False)
    pltpu.sync_copy(acc, out_hbm.at[pl.ds(row0, CPT)])


@functools.partial(
    pl.kernel,
    out_type=(jax.ShapeDtypeStruct((NW * CPT, NB), jnp.float32),
              jax.ShapeDtypeStruct((NB,), jnp.float32)),
    mesh=_MESH,
    compiler_params=_SC_PARAMS,
    scratch_types=[
        pltpu.VMEM((CPT, NB), jnp.float32),
        pltpu.VMEM((CPT, NB), jnp.float32),
        pltpu.VMEM((NB,), jnp.float32),
        pltpu.VMEM((2, 2, CHK), jnp.int32),
        pltpu.SemaphoreType.DMA,
        pltpu.SemaphoreType.DMA,
    ],
)
def _sc_agg_deg(h_hbm, pk_hbm, out_hbm, deg_hbm, h_sl, acc, deg, ebuf,
                sem0, sem1):
    row0, wid = _agg_body(h_hbm, pk_hbm, h_sl, acc, deg, ebuf, sem0, sem1,
                          True)
    pltpu.sync_copy(acc, out_hbm.at[pl.ds(row0, CPT)])

    @pl.when(wid == jnp.int32(0))
    def _():
        pltpu.sync_copy(deg, deg_hbm)


_Z = np.int32(0)
R = 2048
G = NB // R


def _dot(a, b):
    return jnp.dot(a, b, preferred_element_type=jnp.float32,
                   precision=lax.Precision.HIGHEST)


def _tc_encode(h_ref, w_ref, b_ref, out_ref):
    out_ref[...] = lax.dot_general(
        w_ref[...], h_ref[...], (((1,), (1,)), ((), ())),
        preferred_element_type=jnp.float32,
        precision=lax.Precision.HIGHEST) + b_ref[...]


def _tc_mm1(h_ref, agg_ref, deg_ref, wt_ref, wb_ref, b_ref,
            hl_ref, degs_ref, sum_ref, sq_ref):
    degs = jnp.maximum(deg_ref[...], 1.0)
    degs_ref[...] = degs
    aggn = agg_ref[...] * (1.0 / degs)
    hl = _dot(wt_ref[...], h_ref[...]) + _dot(wb_ref[...], aggn) + b_ref[...]
    hl_ref[...] = hl

    @pl.when(pl.program_id(0) == 0)
    def _():
        sum_ref[...] = jnp.zeros_like(sum_ref)
        sq_ref[...] = jnp.zeros_like(sq_ref)

    col0 = pl.program_id(0) * R
    lane = lax.broadcasted_iota(jnp.int32, (1, R), 1) + col0
    hv = hl * (lane < N).astype(jnp.float32)
    sum_ref[...] += jnp.sum(hv, axis=1, keepdims=True)
    sq_ref[...] += jnp.sum(hv * hv, axis=1, keepdims=True)


def _tc_mm(h_ref, agg_ref, degs_ref, wt_ref, wb_ref, b_ref,
           hl_ref, sum_ref, sq_ref):
    aggn = agg_ref[...] * (1.0 / degs_ref[...])
    hl = _dot(wt_ref[...], h_ref[...]) + _dot(wb_ref[...], aggn) + b_ref[...]
    hl_ref[...] = hl

    @pl.when(pl.program_id(0) == 0)
    def _():
        sum_ref[...] = jnp.zeros_like(sum_ref)
        sq_ref[...] = jnp.zeros_like(sq_ref)

    col0 = pl.program_id(0) * R
    lane = lax.broadcasted_iota(jnp.int32, (1, R), 1) + col0
    hv = hl * (lane < N).astype(jnp.float32)
    sum_ref[...] += jnp.sum(hv, axis=1, keepdims=True)
    sq_ref[...] += jnp.sum(hv * hv, axis=1, keepdims=True)


def _bn(hl, sm, sq, g, be):
    mu = sm * (1.0 / N)
    var = sq * (1.0 / N) - mu * mu
    return (hl - mu) * lax.rsqrt(var + 1e-5) * g + be


def _tc_norm(hl_ref, sum_ref, sq_ref, degs_ref, g_ref, be_ref, out_ref):
    hn = _bn(hl_ref[...], sum_ref[...], sq_ref[...], g_ref[...], be_ref[...])
    out_ref[...] = jnp.maximum(hn, 0.0) * lax.rsqrt(degs_ref[...])


def _tc_norm_out(hl_ref, sum_ref, sq_ref, degs_ref, g_ref, be_ref,
                 wo_ref, bo_ref, out_ref):
    hn = _bn(hl_ref[...], sum_ref[...], sq_ref[...], g_ref[...], be_ref[...])
    hr = jnp.maximum(hn, 0.0) * lax.rsqrt(degs_ref[...])
    out_ref[...] = _dot(wo_ref[...], hr) + bo_ref[...]


def kernel(h, edge_index, e, W_enc, b_enc, W_layers, b_layers, gamma, beta,
           W_out, b_out):
    f32 = jnp.float32
    src = edge_index[0].astype(jnp.int32)
    dst = edge_index[1].astype(jnp.int32)
    srcp = jnp.pad(src, (0, EP - E)).reshape(NCH, CHK)
    dstp = jnp.pad(dst, (0, EP - E),
                   constant_values=NB - 1).reshape(NCH, CHK)
    packed = jnp.stack([srcp, dstp], axis=1)

    colsT = pl.BlockSpec((H, R), lambda i: (_Z, i))
    rows1T = pl.BlockSpec((1, R), lambda i: (_Z, i))
    const = pl.BlockSpec((H, H), lambda i: (_Z, _Z))
    ccol = pl.BlockSpec((H, 1), lambda i: (_Z, _Z))
    hrows = pl.BlockSpec((R, D), lambda i: (i, _Z))
    cwenc = pl.BlockSpec((H, D), lambda i: (_Z, _Z))
    cwo = pl.BlockSpec((C, H), lambda i: (_Z, _Z))
    cbo = pl.BlockSpec((C, 1), lambda i: (_Z, _Z))

    h = h.astype(f32)
    hT = pl.pallas_call(
        _tc_encode,
        grid=(G,),
        in_specs=[hrows, cwenc, ccol],
        out_specs=colsT,
        out_shape=jax.ShapeDtypeStruct((H, NB), f32),
    )(h, W_enc.astype(f32).T, b_enc.astype(f32).reshape(H, 1))

    wts = [(W_layers[i, :H, :].astype(f32).T, W_layers[i, H:, :].astype(f32).T,
            b_layers[i].astype(f32).reshape(H, 1),
            gamma[i].astype(f32).reshape(H, 1),
            beta[i].astype(f32).reshape(H, 1)) for i in range(L)]

    aggT, degTf = _sc_agg_deg(hT, packed)
    degT = degTf.reshape(1, NB)
    wt, wb, b, g, be = wts[0]
    hl, degs, sm, sq = pl.pallas_call(
        _tc_mm1,
        grid=(G,),
        in_specs=[colsT, colsT, rows1T, const, const, ccol],
        out_specs=(colsT, rows1T, ccol, ccol),
        out_shape=(jax.ShapeDtypeStruct((H, NB), f32),
                   jax.ShapeDtypeStruct((1, NB), f32),
                   jax.ShapeDtypeStruct((H, 1), f32),
                   jax.ShapeDtypeStruct((H, 1), f32)),
    )(hT, aggT, degT, wt, wb, b)
    hcur = pl.pallas_call(
        _tc_norm,
        grid=(G,),
        in_specs=[colsT, ccol, ccol, rows1T, ccol, ccol],
        out_specs=colsT,
        out_shape=jax.ShapeDtypeStruct((H, NB), f32),
    )(hl, sm, sq, degs, g, be)

    for i in range(1, L):
        aggT = _sc_agg(hcur, packed)
        wt, wb, b, g, be = wts[i]
        hl, sm, sq = pl.pallas_call(
            _tc_mm,
            grid=(G,),
            in_specs=[colsT, colsT, rows1T, const, const, ccol],
            out_specs=(colsT, ccol, ccol),
            out_shape=(jax.ShapeDtypeStruct((H, NB), f32),
                       jax.ShapeDtypeStruct((H, 1), f32),
                       jax.ShapeDtypeStruct((H, 1), f32)),
        )(hcur, aggT, degs, wt, wb, b)
        if i < L - 1:
            hcur = pl.pallas_call(
                _tc_norm,
                grid=(G,),
                in_specs=[colsT, ccol, ccol, rows1T, ccol, ccol],
                out_specs=colsT,
                out_shape=jax.ShapeDtypeStruct((H, NB), f32),
            )(hl, sm, sq, degs, g, be)
        else:
            outT = pl.pallas_call(
                _tc_norm_out,
                grid=(G,),
                in_specs=[colsT, ccol, ccol, rows1T, ccol, ccol, cwo, cbo],
                out_specs=pl.BlockSpec((C, R), lambda i: (_Z, i)),
                out_shape=jax.ShapeDtypeStruct((C, NB), f32),
            )(hl, sm, sq, degs, g, be,
              W_out.astype(f32).T, b_out.astype(f32).reshape(C, 1))
    return outT[:, :N].T.astype(jnp.float64)
